# Initial kernel scaffold; baseline (speedup 1.0000x reference)
import jax
import jax.numpy as jnp
from jax.experimental import pallas as pl


def kernel(x, bn_in_weight, bn_in_bias, bn_in_mean, bn_in_var, bn_out_weight, bn_out_bias, bn_out_mean, bn_out_var, quant_list, context_distribution):
    raise NotImplementedError("write your pallas kernel here")


# speedup vs baseline: 196.8551x; 196.8551x over previous
import functools

import jax
import jax.numpy as jnp
from jax.experimental import pallas as pl
from jax.experimental.pallas import tpu as pltpu

EPS_BN = 1e-5
PAD = 1e30
SAMP = 1024
NQ = 102


def _quantile_kernel(ql_ref, v_ref, out_ref):
    v = v_ref[...]
    sn = jnp.sum((v < 0).astype(jnp.float32), axis=0, keepdims=True)

    def count_body(k, acc):
        row = v_ref[pl.ds(k, 1), :]
        return acc + (row < v).astype(jnp.float32)

    acc = jax.lax.fori_loop(0, SAMP, count_body,
                            jnp.zeros(v.shape, jnp.float32))

    spf = 1002.0 - sn
    nw = 1000.0 / sn
    pw = 1000.0 / spf
    n0 = sn * nw
    total = n0 + spf * pw

    def q_body(j, _):
        t = ql_ref[j] * total
        na = jnp.clip(jnp.floor(t / nw), 0.0, sn)
        pa = jnp.clip(jnp.floor((t - n0) / pw), 0.0, spf)
        above = na + pa
        ac = jnp.minimum(above, 999.0)
        bc = jnp.clip(above - 1.0, 0.0, 999.0)
        wb = jnp.where(bc < sn, (bc + 1.0) * nw, n0 + (bc - sn + 1.0) * pw)
        wa = jnp.where(ac < sn, (ac + 1.0) * nw, n0 + (ac - sn + 1.0) * pw)
        vb = jnp.max(jnp.where(acc <= bc, v, -PAD), axis=0, keepdims=True)
        va = jnp.max(jnp.where(acc <= ac, v, -PAD), axis=0, keepdims=True)
        q = vb + (va - vb) * (t - wb) / (wa - wb + 1e-6)
        out_ref[pl.ds(j, 1), :] = q
        return 0

    jax.lax.fori_loop(0, NQ, q_body, 0)


def _apply_kernel(dq_ref, x_ref, qt_ref, ain_ref, bin_ref, aout_ref,
                  bout_ref, out_ref):
    h = x_ref[...] * ain_ref[...] + bin_ref[...]

    def body(j, acc):
        qrow = qt_ref[pl.ds(j, 1), :]
        return acc + jnp.where(qrow <= h, dq_ref[j], 0.0)

    acc = jax.lax.fori_loop(0, NQ - 1, body, jnp.zeros(h.shape, jnp.float32))
    out_ref[...] = acc * aout_ref[...] + bout_ref[...]


def kernel(x, bn_in_weight, bn_in_bias, bn_in_mean, bn_in_var,
           bn_out_weight, bn_out_bias, bn_out_mean, bn_out_var,
           quant_list, context_distribution):
    B, F = x.shape
    ctx = context_distribution
    N = ctx.shape[1]

    vct = jnp.concatenate([
        ctx.T,
        jnp.full((1, F), -100.0, jnp.float32),
        jnp.full((1, F), 100.0, jnp.float32),
        jnp.full((SAMP - N - 2, F), PAD, jnp.float32),
    ], axis=0)

    a_in = bn_in_weight / jnp.sqrt(bn_in_var + EPS_BN)
    b_in = bn_in_bias - bn_in_mean * a_in
    a_out = bn_out_weight / jnp.sqrt(bn_out_var + EPS_BN)
    b_out = (bn_out_bias - bn_out_mean * a_out) + quant_list[0] * a_out
    dq = quant_list[1:] - quant_list[:-1]

    FB = 256
    qt = pl.pallas_call(
        _quantile_kernel,
        grid=(F // FB,),
        in_specs=[
            pl.BlockSpec(memory_space=pltpu.SMEM),
            pl.BlockSpec((SAMP, FB), lambda i: (0, i)),
        ],
        out_specs=pl.BlockSpec((128, FB), lambda i: (0, i)),
        out_shape=jax.ShapeDtypeStruct((128, F), jnp.float32),
    )(quant_list, vct)

    BB = 512
    out = pl.pallas_call(
        _apply_kernel,
        grid=(B // BB,),
        in_specs=[
            pl.BlockSpec(memory_space=pltpu.SMEM),
            pl.BlockSpec((BB, F), lambda i: (i, 0)),
            pl.BlockSpec((128, F), lambda i: (0, 0)),
            pl.BlockSpec((1, F), lambda i: (0, 0)),
            pl.BlockSpec((1, F), lambda i: (0, 0)),
            pl.BlockSpec((1, F), lambda i: (0, 0)),
            pl.BlockSpec((1, F), lambda i: (0, 0)),
        ],
        out_specs=pl.BlockSpec((BB, F), lambda i: (i, 0)),
        out_shape=jax.ShapeDtypeStruct((B, F), jnp.float32),
    )(dq, x, qt, a_in.reshape(1, F), b_in.reshape(1, F),
      a_out.reshape(1, F), b_out.reshape(1, F))
    return out


# speedup vs baseline: 1102.2167x; 5.5991x over previous
import jax
import jax.numpy as jnp
from jax.experimental import pallas as pl

EPS_BN = 1e-5
BIG = 1e30
NQ = 102
LANES = 1024


def _bitonic_sort_lanes(x):
    n = x.shape[1]
    lane = jax.lax.broadcasted_iota(jnp.int32, (1, n), 1)
    k = 2
    while k <= n:
        d = k // 2
        while d >= 1:
            left = jnp.concatenate([x[:, d:], x[:, :d]], axis=1)
            right = jnp.concatenate([x[:, n - d:], x[:, :n - d]], axis=1)
            low = (lane & d) == 0
            p = jnp.where(low, left, right)
            up = (lane & k) == 0
            take_min = low == up
            x = jnp.where(take_min, jnp.minimum(x, p), jnp.maximum(x, p))
            d //= 2
        k *= 2
    return x


def _gather_sorted(sv, idx):
    local = idx & 127
    chunk = idx >> 7
    res = jnp.zeros(idx.shape, jnp.float32)
    for c in range(8):
        g = jnp.take_along_axis(sv[:, c * 128:(c + 1) * 128], local, axis=1)
        res = jnp.where(chunk == c, g, res)
    return res


def _quantile_kernel(v_ref, ql_ref, out_ref):
    sv = _bitonic_sort_lanes(v_ref[...])
    snf = jnp.sum((sv < 0).astype(jnp.float32), axis=1, keepdims=True)
    spf = 1002.0 - snf
    nw = 1000.0 / snf
    pw = 1000.0 / spf
    n0 = snf * nw
    total = n0 + spf * pw

    t = ql_ref[...] * total
    na = jnp.clip(jnp.floor(t / nw), 0.0, snf)
    pa = jnp.clip(jnp.floor((t - n0) / pw), 0.0, spf)
    above = na + pa
    ac = jnp.minimum(above, 999.0)
    bc = jnp.clip(above - 1.0, 0.0, 999.0)
    wb = jnp.where(bc < snf, (bc + 1.0) * nw, n0 + (bc - snf + 1.0) * pw)
    wa = jnp.where(ac < snf, (ac + 1.0) * nw, n0 + (ac - snf + 1.0) * pw)
    vb = _gather_sorted(sv, bc.astype(jnp.int32))
    va = _gather_sorted(sv, ac.astype(jnp.int32))
    q = vb + (va - vb) * (t - wb) / (wa - wb + 1e-6)
    jlane = jax.lax.broadcasted_iota(jnp.int32, q.shape, 1)
    out_ref[...] = jnp.where(jlane >= NQ, BIG, q)


def _apply_kernel(x_ref, q_ref, ql_ref, ain_ref, bin_ref, aout_ref,
                  bout_ref, out_ref):
    xt = jnp.swapaxes(x_ref[...], 0, 1)
    h = xt * ain_ref[...] + bin_ref[...]
    q = q_ref[...]
    qlt = jnp.broadcast_to(ql_ref[...], q.shape)
    nb = h.shape[1] // 128
    chunks = []
    for c in range(nb):
        hc = h[:, c * 128:(c + 1) * 128]
        cnt = jnp.zeros(hc.shape, jnp.int32)
        for bit in (64, 32, 16, 8, 4, 2, 1):
            t = cnt + (bit - 1)
            qv = jnp.take_along_axis(q, t, axis=1)
            cnt = jnp.where(qv <= hc, t + 1, cnt)
        qidx = jnp.minimum(cnt, NQ - 1)
        chunks.append(jnp.take_along_axis(qlt, qidx, axis=1))
    res = jnp.concatenate(chunks, axis=1)
    res = res * aout_ref[...] + bout_ref[...]
    out_ref[...] = jnp.swapaxes(res, 0, 1)


def kernel(x, bn_in_weight, bn_in_bias, bn_in_mean, bn_in_var,
           bn_out_weight, bn_out_bias, bn_out_mean, bn_out_var,
           quant_list, context_distribution):
    B, F = x.shape
    ctx = context_distribution
    N = ctx.shape[1]

    vc = jnp.concatenate([
        ctx,
        jnp.full((F, 1), -100.0, jnp.float32),
        jnp.full((F, 1), 100.0, jnp.float32),
        jnp.full((F, LANES - N - 2), BIG, jnp.float32),
    ], axis=1)
    ql128 = jnp.concatenate(
        [quant_list, jnp.full((128 - NQ,), 0.99, jnp.float32)]).reshape(1, 128)

    a_in = (bn_in_weight / jnp.sqrt(bn_in_var + EPS_BN)).reshape(F, 1)
    b_in = bn_in_bias.reshape(F, 1) - bn_in_mean.reshape(F, 1) * a_in
    a_out = (bn_out_weight / jnp.sqrt(bn_out_var + EPS_BN)).reshape(F, 1)
    b_out = bn_out_bias.reshape(F, 1) - bn_out_mean.reshape(F, 1) * a_out

    FB = 32
    qt = pl.pallas_call(
        _quantile_kernel,
        grid=(F // FB,),
        in_specs=[
            pl.BlockSpec((FB, LANES), lambda i: (i, 0)),
            pl.BlockSpec((1, 128), lambda i: (0, 0)),
        ],
        out_specs=pl.BlockSpec((FB, 128), lambda i: (i, 0)),
        out_shape=jax.ShapeDtypeStruct((F, 128), jnp.float32),
    )(vc, ql128)

    BB = 128
    out = pl.pallas_call(
        _apply_kernel,
        grid=(B // BB,),
        in_specs=[
            pl.BlockSpec((BB, F), lambda i: (i, 0)),
            pl.BlockSpec((F, 128), lambda i: (0, 0)),
            pl.BlockSpec((1, 128), lambda i: (0, 0)),
            pl.BlockSpec((F, 1), lambda i: (0, 0)),
            pl.BlockSpec((F, 1), lambda i: (0, 0)),
            pl.BlockSpec((F, 1), lambda i: (0, 0)),
            pl.BlockSpec((F, 1), lambda i: (0, 0)),
        ],
        out_specs=pl.BlockSpec((BB, F), lambda i: (i, 0)),
        out_shape=jax.ShapeDtypeStruct((B, F), jnp.float32),
    )(x, qt, ql128, a_in, b_in, a_out, b_out)
    return out


# speedup vs baseline: 1431.4130x; 1.2987x over previous
import jax
import jax.numpy as jnp
from jax.experimental import pallas as pl

EPS_BN = 1e-5
BIG = 1e30
NQ = 102
LANES = 1024


def _bitonic_sort_lanes(x):
    n = x.shape[1]
    lane = jax.lax.broadcasted_iota(jnp.int32, (1, n), 1)
    k = 2
    while k <= n:
        d = k // 2
        while d >= 1:
            left = jnp.concatenate([x[:, d:], x[:, :d]], axis=1)
            right = jnp.concatenate([x[:, n - d:], x[:, :n - d]], axis=1)
            low = (lane & d) == 0
            p = jnp.where(low, left, right)
            up = (lane & k) == 0
            take_min = low == up
            x = jnp.where(take_min, jnp.minimum(x, p), jnp.maximum(x, p))
            d //= 2
        k *= 2
    return x


def _gather_sorted(sv, idx):
    local = idx & 127
    chunk = idx >> 7
    res = jnp.zeros(idx.shape, jnp.float32)
    for c in range(8):
        g = jnp.take_along_axis(sv[:, c * 128:(c + 1) * 128], local, axis=1)
        res = jnp.where(chunk == c, g, res)
    return res


def _quantile_kernel(v_ref, ql_ref, out_ref):
    sv = _bitonic_sort_lanes(v_ref[...])
    snf = jnp.sum((sv < 0).astype(jnp.float32), axis=1, keepdims=True)
    spf = 1002.0 - snf
    nw = 1000.0 / snf
    pw = 1000.0 / spf
    n0 = snf * nw
    total = n0 + spf * pw

    t = ql_ref[...] * total
    na = jnp.clip(jnp.floor(t / nw), 0.0, snf)
    pa = jnp.clip(jnp.floor((t - n0) / pw), 0.0, spf)
    above = na + pa
    ac = jnp.minimum(above, 999.0)
    bc = jnp.clip(above - 1.0, 0.0, 999.0)
    wb = jnp.where(bc < snf, (bc + 1.0) * nw, n0 + (bc - snf + 1.0) * pw)
    wa = jnp.where(ac < snf, (ac + 1.0) * nw, n0 + (ac - snf + 1.0) * pw)
    vb = _gather_sorted(sv, bc.astype(jnp.int32))
    va = _gather_sorted(sv, ac.astype(jnp.int32))
    q = vb + (va - vb) * (t - wb) / (wa - wb + 1e-6)
    jlane = jax.lax.broadcasted_iota(jnp.int32, q.shape, 1)
    out_ref[...] = jnp.where(jlane >= NQ, BIG, q)


def _apply_kernel(x_ref, q_ref, ql_ref, ain_ref, bin_ref, aout_ref,
                  bout_ref, out_ref):
    xt = jnp.swapaxes(x_ref[...], 0, 1)
    h = xt * ain_ref[...] + bin_ref[...]
    q = q_ref[...]
    qlt = jnp.broadcast_to(ql_ref[...], q.shape)
    nb = h.shape[1] // 128
    chunks = []
    for c in range(nb):
        hc = h[:, c * 128:(c + 1) * 128]
        cnt = jnp.zeros(hc.shape, jnp.int32)
        for bit in (64, 32, 16, 8, 4, 2, 1):
            t = cnt + (bit - 1)
            qv = jnp.take_along_axis(q, t, axis=1)
            cnt = jnp.where(qv <= hc, t + 1, cnt)
        qidx = jnp.minimum(cnt, NQ - 1)
        chunks.append(jnp.take_along_axis(qlt, qidx, axis=1))
    res = jnp.concatenate(chunks, axis=1)
    res = res * aout_ref[...] + bout_ref[...]
    out_ref[...] = jnp.swapaxes(res, 0, 1)


def kernel(x, bn_in_weight, bn_in_bias, bn_in_mean, bn_in_var,
           bn_out_weight, bn_out_bias, bn_out_mean, bn_out_var,
           quant_list, context_distribution):
    B, F = x.shape
    ctx = context_distribution
    N = ctx.shape[1]

    vc = jnp.concatenate([
        ctx,
        jnp.full((F, 1), -100.0, jnp.float32),
        jnp.full((F, 1), 100.0, jnp.float32),
        jnp.full((F, LANES - N - 2), BIG, jnp.float32),
    ], axis=1)
    ql128 = jnp.concatenate(
        [quant_list, jnp.full((128 - NQ,), 0.99, jnp.float32)]).reshape(1, 128)

    a_in = (bn_in_weight / jnp.sqrt(bn_in_var + EPS_BN)).reshape(F, 1)
    b_in = bn_in_bias.reshape(F, 1) - bn_in_mean.reshape(F, 1) * a_in
    a_out = (bn_out_weight / jnp.sqrt(bn_out_var + EPS_BN)).reshape(F, 1)
    b_out = bn_out_bias.reshape(F, 1) - bn_out_mean.reshape(F, 1) * a_out

    FB = 64
    qt = pl.pallas_call(
        _quantile_kernel,
        grid=(F // FB,),
        in_specs=[
            pl.BlockSpec((FB, LANES), lambda i: (i, 0)),
            pl.BlockSpec((1, 128), lambda i: (0, 0)),
        ],
        out_specs=pl.BlockSpec((FB, 128), lambda i: (i, 0)),
        out_shape=jax.ShapeDtypeStruct((F, 128), jnp.float32),
    )(vc, ql128)

    BB = 256
    out = pl.pallas_call(
        _apply_kernel,
        grid=(B // BB,),
        in_specs=[
            pl.BlockSpec((BB, F), lambda i: (i, 0)),
            pl.BlockSpec((F, 128), lambda i: (0, 0)),
            pl.BlockSpec((1, 128), lambda i: (0, 0)),
            pl.BlockSpec((F, 1), lambda i: (0, 0)),
            pl.BlockSpec((F, 1), lambda i: (0, 0)),
            pl.BlockSpec((F, 1), lambda i: (0, 0)),
            pl.BlockSpec((F, 1), lambda i: (0, 0)),
        ],
        out_specs=pl.BlockSpec((BB, F), lambda i: (i, 0)),
        out_shape=jax.ShapeDtypeStruct((B, F), jnp.float32),
    )(x, qt, ql128, a_in, b_in, a_out, b_out)
    return out


# speedup vs baseline: 1542.7507x; 1.0778x over previous
import jax
import jax.numpy as jnp
from jax.experimental import pallas as pl

EPS_BN = 1e-5
BIG = 1e30
NQ = 102
LANES = 1024


def _bitonic_sort_lanes(x):
    n = x.shape[1]
    lane = jax.lax.broadcasted_iota(jnp.int32, (1, n), 1)
    k = 2
    while k <= n:
        d = k // 2
        while d >= 1:
            left = jnp.concatenate([x[:, d:], x[:, :d]], axis=1)
            right = jnp.concatenate([x[:, n - d:], x[:, :n - d]], axis=1)
            low = (lane & d) == 0
            p = jnp.where(low, left, right)
            up = (lane & k) == 0
            take_min = low == up
            x = jnp.where(take_min, jnp.minimum(x, p), jnp.maximum(x, p))
            d //= 2
        k *= 2
    return x


def _gather_sorted(sv, idx):
    local = idx & 127
    chunk = idx >> 7
    res = jnp.zeros(idx.shape, jnp.float32)
    for c in range(8):
        g = jnp.take_along_axis(sv[:, c * 128:(c + 1) * 128], local, axis=1)
        res = jnp.where(chunk == c, g, res)
    return res


def _quantile_kernel(v_ref, ql_ref, out_ref):
    sv = _bitonic_sort_lanes(v_ref[...])
    snf = jnp.sum((sv < 0).astype(jnp.float32), axis=1, keepdims=True)
    spf = 1002.0 - snf
    nw = 1000.0 / snf
    pw = 1000.0 / spf
    n0 = snf * nw
    total = n0 + spf * pw

    t = ql_ref[...] * total
    na = jnp.clip(jnp.floor(t / nw), 0.0, snf)
    pa = jnp.clip(jnp.floor((t - n0) / pw), 0.0, spf)
    above = na + pa
    ac = jnp.minimum(above, 999.0)
    bc = jnp.clip(above - 1.0, 0.0, 999.0)
    wb = jnp.where(bc < snf, (bc + 1.0) * nw, n0 + (bc - snf + 1.0) * pw)
    wa = jnp.where(ac < snf, (ac + 1.0) * nw, n0 + (ac - snf + 1.0) * pw)
    vb = _gather_sorted(sv, bc.astype(jnp.int32))
    va = _gather_sorted(sv, ac.astype(jnp.int32))
    q = vb + (va - vb) * (t - wb) / (wa - wb + 1e-6)
    jlane = jax.lax.broadcasted_iota(jnp.int32, q.shape, 1)
    out_ref[...] = jnp.where(jlane >= NQ, BIG, q)


def _apply_kernel(x_ref, q_ref, ql_ref, ain_ref, bin_ref, aout_ref,
                  bout_ref, out_ref):
    xt = jnp.swapaxes(x_ref[...], 0, 1)
    h = xt * ain_ref[...] + bin_ref[...]
    q = q_ref[...]
    qlt = jnp.broadcast_to(ql_ref[...], q.shape)
    nb = h.shape[1] // 128
    chunks = []
    for c in range(nb):
        hc = h[:, c * 128:(c + 1) * 128]
        cnt = jnp.zeros(hc.shape, jnp.int32)
        for bit in (64, 32, 16, 8, 4, 2, 1):
            t = cnt + (bit - 1)
            qv = jnp.take_along_axis(q, t, axis=1)
            cnt = jnp.where(qv <= hc, t + 1, cnt)
        qidx = jnp.minimum(cnt, NQ - 1)
        chunks.append(jnp.take_along_axis(qlt, qidx, axis=1))
    res = jnp.concatenate(chunks, axis=1)
    res = res * aout_ref[...] + bout_ref[...]
    out_ref[...] = jnp.swapaxes(res, 0, 1)


def kernel(x, bn_in_weight, bn_in_bias, bn_in_mean, bn_in_var,
           bn_out_weight, bn_out_bias, bn_out_mean, bn_out_var,
           quant_list, context_distribution):
    B, F = x.shape
    ctx = context_distribution
    N = ctx.shape[1]

    vc = jnp.concatenate([
        ctx,
        jnp.full((F, 1), -100.0, jnp.float32),
        jnp.full((F, 1), 100.0, jnp.float32),
        jnp.full((F, LANES - N - 2), BIG, jnp.float32),
    ], axis=1)
    ql128 = jnp.concatenate(
        [quant_list, jnp.full((128 - NQ,), 0.99, jnp.float32)]).reshape(1, 128)

    a_in = (bn_in_weight / jnp.sqrt(bn_in_var + EPS_BN)).reshape(F, 1)
    b_in = bn_in_bias.reshape(F, 1) - bn_in_mean.reshape(F, 1) * a_in
    a_out = (bn_out_weight / jnp.sqrt(bn_out_var + EPS_BN)).reshape(F, 1)
    b_out = bn_out_bias.reshape(F, 1) - bn_out_mean.reshape(F, 1) * a_out

    FB = 128
    qt = pl.pallas_call(
        _quantile_kernel,
        grid=(F // FB,),
        in_specs=[
            pl.BlockSpec((FB, LANES), lambda i: (i, 0)),
            pl.BlockSpec((1, 128), lambda i: (0, 0)),
        ],
        out_specs=pl.BlockSpec((FB, 128), lambda i: (i, 0)),
        out_shape=jax.ShapeDtypeStruct((F, 128), jnp.float32),
    )(vc, ql128)

    BB = 512
    out = pl.pallas_call(
        _apply_kernel,
        grid=(B // BB,),
        in_specs=[
            pl.BlockSpec((BB, F), lambda i: (i, 0)),
            pl.BlockSpec((F, 128), lambda i: (0, 0)),
            pl.BlockSpec((1, 128), lambda i: (0, 0)),
            pl.BlockSpec((F, 1), lambda i: (0, 0)),
            pl.BlockSpec((F, 1), lambda i: (0, 0)),
            pl.BlockSpec((F, 1), lambda i: (0, 0)),
            pl.BlockSpec((F, 1), lambda i: (0, 0)),
        ],
        out_specs=pl.BlockSpec((BB, F), lambda i: (i, 0)),
        out_shape=jax.ShapeDtypeStruct((B, F), jnp.float32),
    )(x, qt, ql128, a_in, b_in, a_out, b_out)
    return out


# speedup vs baseline: 1588.5295x; 1.0297x over previous
import jax
import jax.numpy as jnp
from jax.experimental import pallas as pl

EPS_BN = 1e-5
BIG = 1e30
NQ = 102
LANES = 1024


def _bitonic_sort_lanes(x):
    n = x.shape[1]
    lane = jax.lax.broadcasted_iota(jnp.int32, (1, n), 1)
    k = 2
    while k <= n:
        d = k // 2
        while d >= 1:
            left = jnp.concatenate([x[:, d:], x[:, :d]], axis=1)
            right = jnp.concatenate([x[:, n - d:], x[:, :n - d]], axis=1)
            low = (lane & d) == 0
            p = jnp.where(low, left, right)
            up = (lane & k) == 0
            take_min = low == up
            x = jnp.where(take_min, jnp.minimum(x, p), jnp.maximum(x, p))
            d //= 2
        k *= 2
    return x


def _gather_sorted(sv, idx):
    local = idx & 127
    chunk = idx >> 7
    res = jnp.zeros(idx.shape, jnp.float32)
    for c in range(8):
        g = jnp.take_along_axis(sv[:, c * 128:(c + 1) * 128], local, axis=1)
        res = jnp.where(chunk == c, g, res)
    return res


def _quantile_table(v, ql):
    sv = _bitonic_sort_lanes(v)
    snf = jnp.sum((sv < 0).astype(jnp.float32), axis=1, keepdims=True)
    spf = 1002.0 - snf
    nw = 1000.0 / snf
    pw = 1000.0 / spf
    n0 = snf * nw
    total = n0 + spf * pw

    t = ql * total
    na = jnp.clip(jnp.floor(t / nw), 0.0, snf)
    pa = jnp.clip(jnp.floor((t - n0) / pw), 0.0, spf)
    above = na + pa
    ac = jnp.minimum(above, 999.0)
    bc = jnp.clip(above - 1.0, 0.0, 999.0)
    wb = jnp.where(bc < snf, (bc + 1.0) * nw, n0 + (bc - snf + 1.0) * pw)
    wa = jnp.where(ac < snf, (ac + 1.0) * nw, n0 + (ac - snf + 1.0) * pw)
    vb = _gather_sorted(sv, bc.astype(jnp.int32))
    va = _gather_sorted(sv, ac.astype(jnp.int32))
    q = vb + (va - vb) * (t - wb) / (wa - wb + 1e-6)
    jlane = jax.lax.broadcasted_iota(jnp.int32, q.shape, 1)
    return jnp.where(jlane >= NQ, BIG, q)


def _fused_kernel(v_ref, ql_ref, x_ref, ain_ref, bin_ref, aout_ref,
                  bout_ref, out_ref):
    q = _quantile_table(v_ref[...], ql_ref[...])

    xt = jnp.swapaxes(x_ref[...], 0, 1)
    h = xt * ain_ref[...] + bin_ref[...]
    qlt = jnp.broadcast_to(ql_ref[...], q.shape)
    nb = h.shape[1] // 128

    def chunk_gather(src, idx):
        return jnp.concatenate(
            [jnp.take_along_axis(src, idx[:, c * 128:(c + 1) * 128], axis=1)
             for c in range(nb)], axis=1)

    cnt = jnp.zeros(h.shape, jnp.int32)
    for bit in (64, 32, 16, 8, 4, 2, 1):
        t = cnt + (bit - 1)
        qv = chunk_gather(q, t)
        cnt = jnp.where(qv <= h, t + 1, cnt)
    res = chunk_gather(qlt, jnp.minimum(cnt, NQ - 1))
    res = res * aout_ref[...] + bout_ref[...]
    out_ref[...] = jnp.swapaxes(res, 0, 1)


def kernel(x, bn_in_weight, bn_in_bias, bn_in_mean, bn_in_var,
           bn_out_weight, bn_out_bias, bn_out_mean, bn_out_var,
           quant_list, context_distribution):
    B, F = x.shape
    ctx = context_distribution
    N = ctx.shape[1]

    vc = jnp.concatenate([
        ctx,
        jnp.full((F, 1), -100.0, jnp.float32),
        jnp.full((F, 1), 100.0, jnp.float32),
        jnp.full((F, LANES - N - 2), BIG, jnp.float32),
    ], axis=1)
    ql128 = jnp.concatenate(
        [quant_list, jnp.full((128 - NQ,), 0.99, jnp.float32)]).reshape(1, 128)

    a_in = (bn_in_weight / jnp.sqrt(bn_in_var + EPS_BN)).reshape(F, 1)
    b_in = bn_in_bias.reshape(F, 1) - bn_in_mean.reshape(F, 1) * a_in
    a_out = (bn_out_weight / jnp.sqrt(bn_out_var + EPS_BN)).reshape(F, 1)
    b_out = bn_out_bias.reshape(F, 1) - bn_out_mean.reshape(F, 1) * a_out

    FB = 128
    out = pl.pallas_call(
        _fused_kernel,
        grid=(F // FB,),
        in_specs=[
            pl.BlockSpec((FB, LANES), lambda i: (i, 0)),
            pl.BlockSpec((1, 128), lambda i: (0, 0)),
            pl.BlockSpec((B, FB), lambda i: (0, i)),
            pl.BlockSpec((FB, 1), lambda i: (i, 0)),
            pl.BlockSpec((FB, 1), lambda i: (i, 0)),
            pl.BlockSpec((FB, 1), lambda i: (i, 0)),
            pl.BlockSpec((FB, 1), lambda i: (i, 0)),
        ],
        out_specs=pl.BlockSpec((B, FB), lambda i: (0, i)),
        out_shape=jax.ShapeDtypeStruct((B, F), jnp.float32),
    )(vc, ql128, x, a_in, b_in, a_out, b_out)
    return out


# speedup vs baseline: 1596.3692x; 1.0049x over previous
import jax
import jax.numpy as jnp
from jax.experimental import pallas as pl

EPS_BN = 1e-5
BIG = 1e30
NQ = 102
LANES = 1024


def _bitonic_sort_lanes(x):
    n = x.shape[1]
    lane = jax.lax.broadcasted_iota(jnp.int32, (1, n), 1)
    k = 2
    while k <= n:
        flip = k < n
        if flip:
            u = jnp.where((lane & k) == 0, 1.0, -1.0)
            x = x * u
        d = k // 2
        while d >= 1:
            left = jnp.concatenate([x[:, d:], x[:, :d]], axis=1)
            right = jnp.concatenate([x[:, n - d:], x[:, :n - d]], axis=1)
            low = (lane & d) == 0
            x = jnp.where(low, jnp.minimum(x, left), jnp.maximum(x, right))
            d //= 2
        if flip:
            x = x * u
        k *= 2
    return x


def _gather_sorted(sv, idx):
    local = idx & 127
    chunk = idx >> 7
    res = jnp.zeros(idx.shape, jnp.float32)
    for c in range(8):
        g = jnp.take_along_axis(sv[:, c * 128:(c + 1) * 128], local, axis=1)
        res = jnp.where(chunk == c, g, res)
    return res


def _quantile_table(v, ql):
    sv = _bitonic_sort_lanes(v)
    snf = jnp.sum((sv < 0).astype(jnp.float32), axis=1, keepdims=True)
    spf = 1002.0 - snf
    nw = 1000.0 / snf
    pw = 1000.0 / spf
    n0 = snf * nw
    total = n0 + spf * pw

    t = ql * total
    na = jnp.clip(jnp.floor(t / nw), 0.0, snf)
    pa = jnp.clip(jnp.floor((t - n0) / pw), 0.0, spf)
    above = na + pa
    ac = jnp.minimum(above, 999.0)
    bc = jnp.clip(above - 1.0, 0.0, 999.0)
    wb = jnp.where(bc < snf, (bc + 1.0) * nw, n0 + (bc - snf + 1.0) * pw)
    wa = jnp.where(ac < snf, (ac + 1.0) * nw, n0 + (ac - snf + 1.0) * pw)
    vb = _gather_sorted(sv, bc.astype(jnp.int32))
    va = _gather_sorted(sv, ac.astype(jnp.int32))
    q = vb + (va - vb) * (t - wb) / (wa - wb + 1e-6)
    jlane = jax.lax.broadcasted_iota(jnp.int32, q.shape, 1)
    return jnp.where(jlane >= NQ, BIG, q)


def _fused_kernel(v_ref, ql_ref, x_ref, ain_ref, bin_ref, aout_ref,
                  bout_ref, out_ref):
    q = _quantile_table(v_ref[...], ql_ref[...])

    xt = jnp.swapaxes(x_ref[...], 0, 1)
    h = xt * ain_ref[...] + bin_ref[...]
    qlt = jnp.broadcast_to(ql_ref[...], q.shape)
    nb = h.shape[1] // 128

    def chunk_gather(src, idx):
        return jnp.concatenate(
            [jnp.take_along_axis(src, idx[:, c * 128:(c + 1) * 128], axis=1)
             for c in range(nb)], axis=1)

    cnt = jnp.zeros(h.shape, jnp.int32)
    for bit in (64, 32, 16, 8, 4, 2, 1):
        t = cnt + (bit - 1)
        qv = chunk_gather(q, t)
        cnt = jnp.where(qv <= h, t + 1, cnt)
    res = chunk_gather(qlt, jnp.minimum(cnt, NQ - 1))
    res = res * aout_ref[...] + bout_ref[...]
    out_ref[...] = jnp.swapaxes(res, 0, 1)


def kernel(x, bn_in_weight, bn_in_bias, bn_in_mean, bn_in_var,
           bn_out_weight, bn_out_bias, bn_out_mean, bn_out_var,
           quant_list, context_distribution):
    B, F = x.shape
    ctx = context_distribution
    N = ctx.shape[1]

    vc = jnp.concatenate([
        ctx,
        jnp.full((F, 1), -100.0, jnp.float32),
        jnp.full((F, 1), 100.0, jnp.float32),
        jnp.full((F, LANES - N - 2), BIG, jnp.float32),
    ], axis=1)
    ql128 = jnp.concatenate(
        [quant_list, jnp.full((128 - NQ,), 0.99, jnp.float32)]).reshape(1, 128)

    a_in = (bn_in_weight / jnp.sqrt(bn_in_var + EPS_BN)).reshape(F, 1)
    b_in = bn_in_bias.reshape(F, 1) - bn_in_mean.reshape(F, 1) * a_in
    a_out = (bn_out_weight / jnp.sqrt(bn_out_var + EPS_BN)).reshape(F, 1)
    b_out = bn_out_bias.reshape(F, 1) - bn_out_mean.reshape(F, 1) * a_out

    FB = 128
    out = pl.pallas_call(
        _fused_kernel,
        grid=(F // FB,),
        in_specs=[
            pl.BlockSpec((FB, LANES), lambda i: (i, 0)),
            pl.BlockSpec((1, 128), lambda i: (0, 0)),
            pl.BlockSpec((B, FB), lambda i: (0, i)),
            pl.BlockSpec((FB, 1), lambda i: (i, 0)),
            pl.BlockSpec((FB, 1), lambda i: (i, 0)),
            pl.BlockSpec((FB, 1), lambda i: (i, 0)),
            pl.BlockSpec((FB, 1), lambda i: (i, 0)),
        ],
        out_specs=pl.BlockSpec((B, FB), lambda i: (0, i)),
        out_shape=jax.ShapeDtypeStruct((B, F), jnp.float32),
    )(vc, ql128, x, a_in, b_in, a_out, b_out)
    return out
